# baseline (speedup 1.0000x reference)
import numpy as np
import jax
import jax.numpy as jnp
from jax.experimental import pallas as pl

_RATIOS = np.array([0.5, 1.0, 2.0])
_SCALES = np.array([1.0, 2.0 ** (1.0 / 3.0), 2.0 ** (2.0 / 3.0)])
_LEVELS = [(64, 64, 8, 32), (32, 32, 16, 64), (16, 16, 32, 128), (8, 8, 64, 256)]
_N_ROWS = sum(h * w * 9 for h, w, _, _ in _LEVELS)
_N_PAD = 49152
_ROW_OFF = [0, 36864, 46080, 48384]


def _box_sizes(box_size):
    anchors = box_size * np.tile(_SCALES, (2, len(_RATIOS))).T
    areas = anchors[:, 0] * anchors[:, 1]
    anchors[:, 0] = np.sqrt(areas * np.repeat(_RATIOS, len(_SCALES)))
    anchors[:, 1] = anchors[:, 0] / np.repeat(_RATIOS, len(_SCALES))
    return anchors.astype(np.float32)


def _sel_by_level(n, vals, dtype):
    out = jnp.full(n.shape, vals[3], dtype)
    for lvl in (2, 1, 0):
        out = jnp.where(n < _ROW_OFF[lvl + 1], jnp.asarray(vals[lvl], dtype), out)
    return out


def _anchor_kernel(out_cols):
    n = jax.lax.broadcasted_iota(jnp.int32, (1, _N_ROWS), 1)
    off = _sel_by_level(n, _ROW_OFF, jnp.int32)
    stride_f = _sel_by_level(n, [float(s) for (_, _, s, _) in _LEVELS], jnp.float32)
    mask_w = _sel_by_level(n, [w - 1 for (_, w, _, _) in _LEVELS], jnp.int32)
    lg_w = _sel_by_level(n, [int(np.log2(w)) for (_, w, _, _) in _LEVELS], jnp.int32)
    size_f = _sel_by_level(n, [float(s) for (_, _, _, s) in _LEVELS], jnp.float32)

    q = n - off
    cell = jnp.floor((q.astype(jnp.float32) + 0.5) * (1.0 / 9.0)).astype(jnp.int32)
    a = q - 9 * cell
    wi = jnp.bitwise_and(cell, mask_w)
    hi = jax.lax.shift_right_logical(cell, lg_w)
    x = (wi.astype(jnp.float32) + 0.5) * stride_f
    y = (hi.astype(jnp.float32) + 0.5) * stride_f

    base = _box_sizes(1.0)
    w = jnp.full(n.shape, float(base[8, 0]), jnp.float32)
    h = jnp.full(n.shape, float(base[8, 1]), jnp.float32)
    for k in range(7, -1, -1):
        sel = a <= k
        w = jnp.where(sel, float(base[k, 0]), w)
        h = jnp.where(sel, float(base[k, 1]), h)
    w = w * size_f
    h = h * size_f

    out_cols[:, :] = jnp.concatenate(
        [x, y, w, h, x - 0.5 * w, y - 0.5 * h, x + 0.5 * w, y + 0.5 * h], axis=0
    )


def kernel(feat_p3, feat_p4, feat_p5, feat_p6):
    del feat_p3, feat_p4, feat_p5, feat_p6
    big = pl.pallas_call(
        _anchor_kernel,
        out_shape=jax.ShapeDtypeStruct((8, _N_ROWS), jnp.float32),
    )()
    return big[0:4].T, big[4:8].T


# speedup vs baseline: 2.1296x; 2.1296x over previous
import numpy as np
import jax
import jax.numpy as jnp
from jax.experimental import pallas as pl

_RATIOS = np.array([0.5, 1.0, 2.0])
_SCALES = np.array([1.0, 2.0 ** (1.0 / 3.0), 2.0 ** (2.0 / 3.0)])
_LEVELS = [(64, 64, 8, 32), (32, 32, 16, 64), (16, 16, 32, 128), (8, 8, 64, 256)]
_N_ROWS = sum(h * w * 9 for h, w, _, _ in _LEVELS)
_N_PAD = 49152
_ROW_OFF = [0, 36864, 46080, 48384]


def _box_sizes(box_size):
    anchors = box_size * np.tile(_SCALES, (2, len(_RATIOS))).T
    areas = anchors[:, 0] * anchors[:, 1]
    anchors[:, 0] = np.sqrt(areas * np.repeat(_RATIOS, len(_SCALES)))
    anchors[:, 1] = anchors[:, 0] / np.repeat(_RATIOS, len(_SCALES))
    return anchors.astype(np.float32)


def _anchor_kernel(out_cols, out_cols2):
    n = jax.lax.broadcasted_iota(jnp.int32, (1, _N_ROWS), 1)
    c1 = n >= _ROW_OFF[1]
    c2 = n >= _ROW_OFF[2]
    c3 = n >= _ROW_OFF[3]
    off = (
        jnp.where(c1, _ROW_OFF[1], 0)
        + jnp.where(c2, _ROW_OFF[2] - _ROW_OFF[1], 0)
        + jnp.where(c3, _ROW_OFF[3] - _ROW_OFF[2], 0)
    )
    stride_f = jnp.where(c1, jnp.where(c2, jnp.where(c3, 64.0, 32.0), 16.0), 8.0)
    size_f = stride_f * 4.0
    mask_w = jnp.where(c1, jnp.where(c2, jnp.where(c3, 7, 15), 31), 63)
    lg_w = jnp.where(c1, jnp.where(c2, jnp.where(c3, 3, 4), 5), 6)

    q = n - off
    cell = jnp.floor((q.astype(jnp.float32) + 0.5) * (1.0 / 9.0)).astype(jnp.int32)
    a = q - 9 * cell
    wi = jnp.bitwise_and(cell, mask_w)
    hi = jax.lax.shift_right_logical(cell, lg_w)
    x = (wi.astype(jnp.float32) + 0.5) * stride_f
    y = (hi.astype(jnp.float32) + 0.5) * stride_f

    af = a.astype(jnp.float32)
    jf = jnp.floor((af + 0.5) * (1.0 / 3.0))
    u = (af - 3.0 * jf) * (1.0 / 3.0)
    v = (jf - 1.0) * 0.5
    w = jnp.exp2(u + v) * size_f
    h = jnp.exp2(u - v) * size_f

    out_cols[:, :] = jnp.concatenate([x, y, w, h], axis=0)
    out_cols2[:, :] = jnp.concatenate(
        [x - 0.5 * w, y - 0.5 * h, x + 0.5 * w, y + 0.5 * h], axis=0
    )


def kernel(feat_p3, feat_p4, feat_p5, feat_p6):
    del feat_p3, feat_p4, feat_p5, feat_p6
    cols = jax.ShapeDtypeStruct((4, _N_ROWS), jnp.float32)
    big0, big1 = pl.pallas_call(
        _anchor_kernel,
        out_shape=(cols, cols),
    )()
    return big0.T, big1.T


# speedup vs baseline: 2.1466x; 1.0080x over previous
import numpy as np
import jax
import jax.numpy as jnp
from jax.experimental import pallas as pl

_RATIOS = np.array([0.5, 1.0, 2.0])
_SCALES = np.array([1.0, 2.0 ** (1.0 / 3.0), 2.0 ** (2.0 / 3.0)])
_LEVELS = [(64, 64, 8, 32), (32, 32, 16, 64), (16, 16, 32, 128), (8, 8, 64, 256)]
_N_ROWS = sum(h * w * 9 for h, w, _, _ in _LEVELS)
_N_PAD = 49152
_ROW_OFF = [0, 36864, 46080, 48384]


def _box_sizes(box_size):
    anchors = box_size * np.tile(_SCALES, (2, len(_RATIOS))).T
    areas = anchors[:, 0] * anchors[:, 1]
    anchors[:, 0] = np.sqrt(areas * np.repeat(_RATIOS, len(_SCALES)))
    anchors[:, 1] = anchors[:, 0] / np.repeat(_RATIOS, len(_SCALES))
    return anchors.astype(np.float32)


def _anchor_kernel(out_cols, out_cols2):
    for lvl, (H, W, stride, size) in enumerate(_LEVELS):
        rows = H * W * 9
        seg = _ROW_OFF[lvl]
        q = jax.lax.broadcasted_iota(jnp.int32, (1, rows), 1)
        cell = jnp.floor((q.astype(jnp.float32) + 0.5) * (1.0 / 9.0)).astype(jnp.int32)
        a = q - 9 * cell
        wi = jnp.bitwise_and(cell, W - 1)
        hi = jax.lax.shift_right_logical(cell, int(np.log2(W)))
        x = (wi.astype(jnp.float32) + 0.5) * float(stride)
        y = (hi.astype(jnp.float32) + 0.5) * float(stride)

        af = a.astype(jnp.float32)
        jf = jnp.floor((af + 0.5) * (1.0 / 3.0))
        u = (af - 3.0 * jf) * (1.0 / 3.0)
        v = (jf - 1.0) * 0.5
        w = jnp.exp2(u + v) * float(size)
        h = jnp.exp2(u - v) * float(size)

        out_cols[:, seg:seg + rows] = jnp.concatenate([x, y, w, h], axis=0)
        out_cols2[:, seg:seg + rows] = jnp.concatenate(
            [x - 0.5 * w, y - 0.5 * h, x + 0.5 * w, y + 0.5 * h], axis=0
        )


def kernel(feat_p3, feat_p4, feat_p5, feat_p6):
    del feat_p3, feat_p4, feat_p5, feat_p6
    cols = jax.ShapeDtypeStruct((4, _N_ROWS), jnp.float32)
    big0, big1 = pl.pallas_call(
        _anchor_kernel,
        out_shape=(cols, cols),
    )()
    return big0.T, big1.T
